# baseline (speedup 1.0000x reference)
import functools

import jax
import jax.numpy as jnp
from jax import lax
from jax.experimental import pallas as pl
from jax.experimental.pallas import tpu as pltpu
from jax.experimental.pallas import tpu_sc as plsc

_D = 64
_SUB = 128
_K = 2
_CHUNK = _SUB * _K
_NBUF = 2


def _body(n_pairs, n_workers, table_hbm, idxy_hbm, idxx_hbm, out_hbm,
          table_sh, idxy_v, idxx_v, ybuf, xbuf, sem_tab, sem_idx, sem_gat,
          sem_out):
    pairs_per_w = n_pairs // n_workers
    n_chunks = pairs_per_w // _CHUNK
    cid = lax.axis_index("c")
    sid = lax.axis_index("s")
    wid = sid * 2 + cid
    base_w = wid * pairs_per_w

    @pl.when(sid == 0)
    def _():
        pltpu.async_copy(table_hbm, table_sh, sem_tab).wait()
    plsc.subcore_barrier()

    def idx_srcs(i):
        r0 = (base_w + i * _CHUNK) // _SUB
        return idxy_hbm.at[pl.ds(r0, _K)], idxx_hbm.at[pl.ds(r0, _K)]

    def out_dst_y(i):
        return out_hbm.at[pl.ds(base_w + i * _CHUNK, _CHUNK), pl.ds(0, _D)]

    def out_dst_x(i):
        return out_hbm.at[pl.ds(base_w + i * _CHUNK, _CHUNK), pl.ds(_D, _D)]

    def start_idx(i, slot):
        ysrc, xsrc = idx_srcs(i)
        pltpu.async_copy(ysrc, idxy_v.at[slot], sem_idx.at[slot])
        pltpu.async_copy(xsrc, idxx_v.at[slot], sem_idx.at[slot])

    def wait_idx(i, slot):
        ysrc, xsrc = idx_srcs(i)
        pltpu.make_async_copy(ysrc, idxy_v.at[slot], sem_idx.at[slot]).wait()
        pltpu.make_async_copy(xsrc, idxx_v.at[slot], sem_idx.at[slot]).wait()

    start_idx(0, 0)

    def body(i, _):
        b = lax.rem(i, _NBUF)
        nb = lax.rem(i + 1, _NBUF)
        wait_idx(i, b)
        start_idx(lax.min(i + 1, n_chunks - 1), nb)

        @pl.when(i >= _NBUF)
        def _():
            pltpu.make_async_copy(ybuf.at[b], out_dst_y(i), sem_out.at[b]).wait()
            pltpu.make_async_copy(xbuf.at[b], out_dst_x(i), sem_out.at[b]).wait()

        for j in range(_K):
            rs = pl.ds(j * _SUB, _SUB)
            pltpu.async_copy(table_hbm.at[idxy_v.at[b, j]],
                             ybuf.at[b, rs], sem_gat)
            pltpu.async_copy(table_hbm.at[idxx_v.at[b, j]],
                             xbuf.at[b, rs], sem_gat)
        for j in range(_K):
            rs = pl.ds(j * _SUB, _SUB)
            pltpu.make_async_copy(table_hbm.at[idxy_v.at[b, j]],
                                  ybuf.at[b, rs], sem_gat).wait()
            pltpu.make_async_copy(table_hbm.at[idxx_v.at[b, j]],
                                  xbuf.at[b, rs], sem_gat).wait()

        pltpu.async_copy(ybuf.at[b], out_dst_y(i), sem_out.at[b])
        pltpu.async_copy(xbuf.at[b], out_dst_x(i), sem_out.at[b])
        return ()

    lax.fori_loop(0, n_chunks, body, ())

    wait_idx(n_chunks - 1, n_chunks % _NBUF)
    for b in range(_NBUF):
        pltpu.make_async_copy(ybuf.at[b], out_dst_y(0), sem_out.at[b]).wait()
        pltpu.make_async_copy(xbuf.at[b], out_dst_x(0), sem_out.at[b]).wait()


def kernel(pe, coords):
    B, S, _ = coords.shape
    n_pairs = B * S
    idx_y = coords[..., 1].reshape(n_pairs // _SUB, _SUB).astype(jnp.int32)
    idx_x = coords[..., 0].reshape(n_pairs // _SUB, _SUB).astype(jnp.int32)

    n_workers = 32

    mesh = plsc.VectorSubcoreMesh(core_axis_name="c", subcore_axis_name="s")
    run = pl.kernel(
        functools.partial(_body, n_pairs, n_workers),
        out_type=jax.ShapeDtypeStruct((n_pairs, 2 * _D), jnp.float32),
        mesh=mesh,
        scratch_types=[
            pltpu.VMEM_SHARED((500, _D), jnp.float32),
            pltpu.VMEM((_NBUF, _K, _SUB), jnp.int32),
            pltpu.VMEM((_NBUF, _K, _SUB), jnp.int32),
            pltpu.VMEM((_NBUF, _CHUNK, _D), jnp.float32),
            pltpu.VMEM((_NBUF, _CHUNK, _D), jnp.float32),
            pltpu.SemaphoreType.DMA,
            pltpu.SemaphoreType.DMA((_NBUF,)),
            pltpu.SemaphoreType.DMA,
            pltpu.SemaphoreType.DMA((_NBUF,)),
        ],
        compiler_params=pltpu.CompilerParams(use_tc_tiling_on_sc=False),
    )
    out = run(pe, idx_y, idx_x)
    return out.reshape(B, S, 2 * _D)


# speedup vs baseline: 3.7575x; 3.7575x over previous
import functools

import jax
import jax.numpy as jnp
from jax import lax
from jax.experimental import pallas as pl
from jax.experimental.pallas import tpu as pltpu
from jax.experimental.pallas import tpu_sc as plsc

_D = 64
_SUB = 128
_K = 2
_CHUNK = _SUB * _K
_NBUF = 2


def _body(n_pairs, n_workers, table_hbm, idxy_hbm, idxx_hbm, out_hbm,
          table_sh, idxy_v, idxx_v, ybuf, xbuf, sem_tab, sem_idx, sem_gat,
          sem_out):
    pairs_per_w = n_pairs // n_workers
    n_chunks = pairs_per_w // _CHUNK
    cid = lax.axis_index("c")
    sid = lax.axis_index("s")
    wid = sid * 2 + cid
    base_w = wid * pairs_per_w

    @pl.when(sid == 0)
    def _():
        pltpu.async_copy(table_hbm, table_sh, sem_tab).wait()
    plsc.subcore_barrier()

    def idx_srcs(i):
        r0 = (base_w + i * _CHUNK) // _SUB
        return idxy_hbm.at[pl.ds(r0, _K)], idxx_hbm.at[pl.ds(r0, _K)]

    def out_dst_y(i):
        return out_hbm.at[pl.ds(base_w + i * _CHUNK, _CHUNK), pl.ds(0, _D)]

    def out_dst_x(i):
        return out_hbm.at[pl.ds(base_w + i * _CHUNK, _CHUNK), pl.ds(_D, _D)]

    def start_idx(i, slot):
        ysrc, xsrc = idx_srcs(i)
        pltpu.async_copy(ysrc, idxy_v.at[slot], sem_idx.at[slot])
        pltpu.async_copy(xsrc, idxx_v.at[slot], sem_idx.at[slot])

    def wait_idx(i, slot):
        ysrc, xsrc = idx_srcs(i)
        pltpu.make_async_copy(ysrc, idxy_v.at[slot], sem_idx.at[slot]).wait()
        pltpu.make_async_copy(xsrc, idxx_v.at[slot], sem_idx.at[slot]).wait()

    start_idx(0, 0)

    def body(i, _):
        b = lax.rem(i, _NBUF)
        nb = lax.rem(i + 1, _NBUF)
        wait_idx(i, b)
        start_idx(lax.min(i + 1, n_chunks - 1), nb)

        @pl.when(i >= _NBUF)
        def _():
            pltpu.make_async_copy(ybuf.at[b], out_dst_y(i), sem_out.at[b]).wait()
            pltpu.make_async_copy(xbuf.at[b], out_dst_x(i), sem_out.at[b]).wait()

        for j in range(_K):
            rs = pl.ds(j * _SUB, _SUB)
            pltpu.async_copy(table_sh.at[idxy_v.at[b, j]],
                             ybuf.at[b, rs], sem_gat)
            pltpu.async_copy(table_sh.at[idxx_v.at[b, j]],
                             xbuf.at[b, rs], sem_gat)
        for j in range(_K):
            rs = pl.ds(j * _SUB, _SUB)
            pltpu.make_async_copy(table_sh.at[idxy_v.at[b, j]],
                                  ybuf.at[b, rs], sem_gat).wait()
            pltpu.make_async_copy(table_sh.at[idxx_v.at[b, j]],
                                  xbuf.at[b, rs], sem_gat).wait()

        pltpu.async_copy(ybuf.at[b], out_dst_y(i), sem_out.at[b])
        pltpu.async_copy(xbuf.at[b], out_dst_x(i), sem_out.at[b])
        return ()

    lax.fori_loop(0, n_chunks, body, ())

    wait_idx(n_chunks - 1, n_chunks % _NBUF)
    for b in range(_NBUF):
        pltpu.make_async_copy(ybuf.at[b], out_dst_y(0), sem_out.at[b]).wait()
        pltpu.make_async_copy(xbuf.at[b], out_dst_x(0), sem_out.at[b]).wait()


def kernel(pe, coords):
    B, S, _ = coords.shape
    n_pairs = B * S
    idx_y = coords[..., 1].reshape(n_pairs // _SUB, _SUB).astype(jnp.int32)
    idx_x = coords[..., 0].reshape(n_pairs // _SUB, _SUB).astype(jnp.int32)

    n_workers = 32

    mesh = plsc.VectorSubcoreMesh(core_axis_name="c", subcore_axis_name="s")
    run = pl.kernel(
        functools.partial(_body, n_pairs, n_workers),
        out_type=jax.ShapeDtypeStruct((n_pairs, 2 * _D), jnp.float32),
        mesh=mesh,
        scratch_types=[
            pltpu.VMEM_SHARED((500, _D), jnp.float32),
            pltpu.VMEM((_NBUF, _K, _SUB), jnp.int32),
            pltpu.VMEM((_NBUF, _K, _SUB), jnp.int32),
            pltpu.VMEM((_NBUF, _CHUNK, _D), jnp.float32),
            pltpu.VMEM((_NBUF, _CHUNK, _D), jnp.float32),
            pltpu.SemaphoreType.DMA,
            pltpu.SemaphoreType.DMA((_NBUF,)),
            pltpu.SemaphoreType.DMA,
            pltpu.SemaphoreType.DMA((_NBUF,)),
        ],
        compiler_params=pltpu.CompilerParams(use_tc_tiling_on_sc=False),
    )
    out = run(pe, idx_y, idx_x)
    return out.reshape(B, S, 2 * _D)
